# baseline (speedup 1.0000x reference)
import functools

import jax
import jax.numpy as jnp
from jax import lax
from jax.experimental import pallas as pl
from jax.experimental.pallas import tpu as pltpu
from jax.experimental.pallas import tpu_sc as plsc

NC = 2
NS = 16
L = 16
NW = NC * NS
BATCH = 16384
DIM = 64
BLK = 8
PAIRS = DIM // 2
NBLOCKS = 1000000 // BLK
RPW = BATCH // NW
CHUNK = 16
NCHUNK = RPW // CHUNK

_mesh = plsc.VectorSubcoreMesh(
    core_axis_name="c", subcore_axis_name="s", num_cores=NC, num_subcores=NS
)


@functools.partial(
    pl.kernel,
    out_type=jax.ShapeDtypeStruct((BATCH,), jnp.float32),
    mesh=_mesh,
    compiler_params=pltpu.CompilerParams(needs_layout_passes=False),
    scratch_types=[
        pltpu.VMEM((RPW,), jnp.int32),
        pltpu.VMEM((RPW,), jnp.int32),
        pltpu.VMEM((CHUNK, BLK, PAIRS), jnp.int32),
        pltpu.VMEM((CHUNK, BLK, PAIRS), jnp.int32),
        pltpu.VMEM((CHUNK, BLK, PAIRS), jnp.int32),
        pltpu.VMEM((CHUNK, BLK, PAIRS), jnp.int32),
        pltpu.VMEM((RPW,), jnp.float32),
        pltpu.SemaphoreType.DMA,
        pltpu.SemaphoreType.DMA,
        pltpu.SemaphoreType.DMA,
        pltpu.SemaphoreType.DMA,
    ],
)
def _bprmf_sc(u_hbm, i_hbm, uw_hbm, iw_hbm, out_hbm,
              uraw, iraw, ublk_a, iblk_a, ublk_b, iblk_b, outv,
              sem_ua, sem_ia, sem_ub, sem_ib):
    wid = lax.axis_index("s") * NC + lax.axis_index("c")
    base = wid * RPW

    pltpu.sync_copy(u_hbm.at[pl.ds(base, RPW)], uraw)
    pltpu.sync_copy(i_hbm.at[pl.ds(base, RPW)], iraw)

    def issue(c, ublk, iblk, sem_u, sem_i):
        for g in range(CHUNK // L):
            sl = pl.ds(c * CHUNK + g * L, L)
            ubv = lax.shift_right_logical(uraw[sl], 3)
            ibv = lax.shift_right_logical(iraw[sl], 3)
            for s in range(L):
                slot = g * L + s
                pltpu.async_copy(uw_hbm.at[ubv[s]], ublk.at[slot], sem_u)
                pltpu.async_copy(iw_hbm.at[ibv[s]], iblk.at[slot], sem_i)

    def drain(ublk, iblk, sem_u, sem_i):
        dummy = pl.ds(0, CHUNK)
        pltpu.make_async_copy(uw_hbm.at[dummy], ublk, sem_u).wait()
        pltpu.make_async_copy(iw_hbm.at[dummy], iblk, sem_i).wait()

    def compute(c, ublk, iblk):
        for g in range(CHUNK // L):
            sl = pl.ds(c * CHUNK + g * L, L)
            ur = jnp.bitwise_and(uraw[sl], 7)
            ir = jnp.bitwise_and(iraw[sl], 7)
            gslots = lax.iota(jnp.int32, L) + g * L
            acc = jnp.zeros((L,), jnp.float32)
            himask = jnp.full((L,), -65536, jnp.int32)
            for pp in range(PAIRS):
                pv = jnp.full((L,), pp, jnp.int32)
                uv = plsc.load_gather(ublk, [gslots, ur, pv])
                iv = plsc.load_gather(iblk, [gslots, ir, pv])
                u_lo = lax.bitcast_convert_type(
                    lax.shift_left(uv, 16), jnp.float32)
                i_lo = lax.bitcast_convert_type(
                    lax.shift_left(iv, 16), jnp.float32)
                u_hi = lax.bitcast_convert_type(
                    jnp.bitwise_and(uv, himask), jnp.float32)
                i_hi = lax.bitcast_convert_type(
                    jnp.bitwise_and(iv, himask), jnp.float32)
                acc = acc + u_lo * i_lo + u_hi * i_hi
            outv[pl.ds(c * CHUNK + g * L, L)] = acc

    issue(0, ublk_a, iblk_a, sem_ua, sem_ia)

    def pair_body(j, carry):
        c = j * 2
        issue(c + 1, ublk_b, iblk_b, sem_ub, sem_ib)
        drain(ublk_a, iblk_a, sem_ua, sem_ia)
        compute(c, ublk_a, iblk_a)

        @pl.when(j < NCHUNK // 2 - 1)
        def _():
            issue(c + 2, ublk_a, iblk_a, sem_ua, sem_ia)

        drain(ublk_b, iblk_b, sem_ub, sem_ib)
        compute(c + 1, ublk_b, iblk_b)
        return carry

    lax.fori_loop(0, NCHUNK // 2, pair_body, 0)

    pltpu.sync_copy(outv, out_hbm.at[pl.ds(base, RPW)])


def _pack(w):
    wb = w.astype(jnp.bfloat16).reshape(1000000, PAIRS, 2)
    wi = jax.lax.bitcast_convert_type(wb, jnp.int32)
    return wi.reshape(NBLOCKS, BLK, PAIRS)


def kernel(u, i, user_weight, item_weight):
    return _bprmf_sc(u.astype(jnp.int32), i.astype(jnp.int32),
                     _pack(user_weight), _pack(item_weight))


# speedup vs baseline: 5.4115x; 5.4115x over previous
import functools

import jax
import jax.numpy as jnp
from jax import lax
from jax.experimental import pallas as pl
from jax.experimental.pallas import tpu as pltpu
from jax.experimental.pallas import tpu_sc as plsc

NC = 2
NS = 16
L = 16
NW = NC * NS
BATCH = 16384
DIM = 64
BLK = 8
NBLOCKS = 1000000 // BLK
RPW = BATCH // NW
CHUNK = 16
NCHUNK = RPW // CHUNK

_mesh = plsc.VectorSubcoreMesh(
    core_axis_name="c", subcore_axis_name="s", num_cores=NC, num_subcores=NS
)


@functools.partial(
    pl.kernel,
    out_type=jax.ShapeDtypeStruct((BATCH,), jnp.float32),
    mesh=_mesh,
    compiler_params=pltpu.CompilerParams(needs_layout_passes=False),
    scratch_types=[
        pltpu.VMEM((RPW,), jnp.int32),
        pltpu.VMEM((RPW,), jnp.int32),
        pltpu.VMEM((CHUNK, BLK, DIM), jnp.float32),
        pltpu.VMEM((CHUNK, BLK, DIM), jnp.float32),
        pltpu.VMEM((CHUNK, BLK, DIM), jnp.float32),
        pltpu.VMEM((CHUNK, BLK, DIM), jnp.float32),
        pltpu.VMEM((RPW,), jnp.float32),
        pltpu.SemaphoreType.DMA,
        pltpu.SemaphoreType.DMA,
        pltpu.SemaphoreType.DMA,
        pltpu.SemaphoreType.DMA,
    ],
)
def _bprmf_sc(u_hbm, i_hbm, uw_hbm, iw_hbm, out_hbm,
              uraw, iraw, ublk_a, iblk_a, ublk_b, iblk_b, outv,
              sem_ua, sem_ia, sem_ub, sem_ib):
    wid = lax.axis_index("s") * NC + lax.axis_index("c")
    base = wid * RPW

    pltpu.sync_copy(u_hbm.at[pl.ds(base, RPW)], uraw)
    pltpu.sync_copy(i_hbm.at[pl.ds(base, RPW)], iraw)

    def issue(c, ublk, iblk, sem_u, sem_i):
        for g in range(CHUNK // L):
            sl = pl.ds(c * CHUNK + g * L, L)
            ubv = lax.shift_right_logical(uraw[sl], 3)
            ibv = lax.shift_right_logical(iraw[sl], 3)
            for s in range(L):
                slot = g * L + s
                pltpu.async_copy(uw_hbm.at[ubv[s]], ublk.at[slot], sem_u)
                pltpu.async_copy(iw_hbm.at[ibv[s]], iblk.at[slot], sem_i)

    def drain(ublk, iblk, sem_u, sem_i):
        dummy = pl.ds(0, CHUNK)
        pltpu.make_async_copy(uw_hbm.at[dummy], ublk, sem_u).wait()
        pltpu.make_async_copy(iw_hbm.at[dummy], iblk, sem_i).wait()

    def compute(c, ublk, iblk):
        for g in range(CHUNK // L):
            sl = pl.ds(c * CHUNK + g * L, L)
            ur = jnp.bitwise_and(uraw[sl], 7)
            ir = jnp.bitwise_and(iraw[sl], 7)
            gslots = lax.iota(jnp.int32, L) + g * L
            acc = jnp.zeros((L,), jnp.float32)
            for f in range(DIM):
                fv = jnp.full((L,), f, jnp.int32)
                uv = plsc.load_gather(ublk, [gslots, ur, fv])
                iv = plsc.load_gather(iblk, [gslots, ir, fv])
                acc = acc + uv * iv
            outv[pl.ds(c * CHUNK + g * L, L)] = acc

    issue(0, ublk_a, iblk_a, sem_ua, sem_ia)

    def pair_body(j, carry):
        c = j * 2
        issue(c + 1, ublk_b, iblk_b, sem_ub, sem_ib)
        drain(ublk_a, iblk_a, sem_ua, sem_ia)
        compute(c, ublk_a, iblk_a)

        @pl.when(j < NCHUNK // 2 - 1)
        def _():
            issue(c + 2, ublk_a, iblk_a, sem_ua, sem_ia)

        drain(ublk_b, iblk_b, sem_ub, sem_ib)
        compute(c + 1, ublk_b, iblk_b)
        return carry

    lax.fori_loop(0, NCHUNK // 2, pair_body, 0)

    pltpu.sync_copy(outv, out_hbm.at[pl.ds(base, RPW)])


def kernel(u, i, user_weight, item_weight):
    uw3 = jnp.reshape(user_weight, (NBLOCKS, BLK, DIM))
    iw3 = jnp.reshape(item_weight, (NBLOCKS, BLK, DIM))
    return _bprmf_sc(u.astype(jnp.int32), i.astype(jnp.int32), uw3, iw3)
